# Initial kernel scaffold; baseline (speedup 1.0000x reference)
import jax
import jax.numpy as jnp
from jax.experimental import pallas as pl


def kernel(idx, targets, table):
    raise NotImplementedError("write your pallas kernel here")


# speedup vs baseline: 1.4508x; 1.4508x over previous
import functools

import jax
import jax.numpy as jnp
from jax import lax
from jax.experimental import pallas as pl
from jax.experimental.pallas import tpu as pltpu
from jax.experimental.pallas import tpu_sc as plsc

_L = 16
_NW = 32


def _lse_body(t_ref, o_ref):
    x = t_ref[...]
    m = jnp.max(x, axis=1)
    s = jnp.sum(jnp.exp(x - m[:, None]), axis=1)
    o_ref[...] = m + jnp.log(s)


def _fin_body(inv_n, p_ref, o_ref):
    o_ref[...] = (jnp.sum(p_ref[...]) * inv_n).reshape(1, 1)


def _sc_gather_loss(table, idx_flat, tgt_flat, lse_table, n, v, c):
    b_per_w = n // _NW
    ch = 64
    n_chunks = b_per_w // ch
    mesh = plsc.VectorSubcoreMesh(core_axis_name="c", subcore_axis_name="s")

    @functools.partial(
        pl.kernel,
        out_type=[
            jax.ShapeDtypeStruct((n, c), jnp.float32),
            jax.ShapeDtypeStruct((_NW, _L), jnp.float32),
        ],
        mesh=mesh,
        compiler_params=pltpu.CompilerParams(
            needs_layout_passes=False, use_tc_tiling_on_sc=False),
        scratch_types=[
            pltpu.VMEM((b_per_w,), jnp.int32),
            pltpu.VMEM((b_per_w,), jnp.int32),
            pltpu.VMEM((v,), jnp.float32),
            pltpu.VMEM((ch, c), jnp.float32),
            pltpu.VMEM((_L,), jnp.float32),
            pltpu.SemaphoreType.DMA,
        ],
    )
    def sc_kernel(table_hbm, idx_hbm, tgt_hbm, lse_hbm, out_hbm, part_hbm,
                  idx_v, tgt_v, lse_v, rows_v, acc_v, sem):
        wid = lax.axis_index("s") * 2 + lax.axis_index("c")
        base = wid * b_per_w
        pltpu.sync_copy(idx_hbm.at[pl.ds(base, b_per_w)], idx_v)
        pltpu.sync_copy(tgt_hbm.at[pl.ds(base, b_per_w)], tgt_v)
        pltpu.sync_copy(lse_hbm, lse_v)

        def chunk_body(g, acc):
            off = g * ch
            pltpu.async_copy(
                table_hbm.at[idx_v.at[pl.ds(off, ch)]], rows_v, sem).wait()
            pltpu.sync_copy(rows_v, out_hbm.at[pl.ds(base + off, ch)])

            def inner(j, a):
                sl = pl.ds(off + j * _L, _L)
                i16 = idx_v[sl]
                t16 = tgt_v[sl]
                lse16 = plsc.load_gather(lse_v, [i16])
                r16 = lax.iota(jnp.int32, _L) + j * _L
                p16 = plsc.load_gather(rows_v, [r16, t16])
                return a + (lse16 - p16)

            return lax.fori_loop(0, ch // _L, inner, acc)

        acc = lax.fori_loop(0, n_chunks, chunk_body, jnp.zeros((_L,), jnp.float32))
        acc_v[...] = acc
        pltpu.sync_copy(acc_v, part_hbm.at[wid])

    return sc_kernel(table, idx_flat, tgt_flat, lse_table)


def kernel(idx, targets, table):
    b, t = idx.shape
    v, c = table.shape
    n = b * t
    idx_flat = idx.reshape(-1)
    tgt_flat = targets.reshape(-1)

    lse_table = pl.pallas_call(
        _lse_body,
        out_shape=jax.ShapeDtypeStruct((v,), jnp.float32),
    )(table)

    logits_flat, partials = _sc_gather_loss(
        table, idx_flat, tgt_flat, lse_table, n, v, c)

    loss2d = pl.pallas_call(
        functools.partial(_fin_body, 1.0 / n),
        out_shape=jax.ShapeDtypeStruct((1, 1), jnp.float32),
    )(partials)

    return logits_flat, loss2d[0, 0]
